# Initial kernel scaffold; baseline (speedup 1.0000x reference)
import jax
import jax.numpy as jnp
from jax.experimental import pallas as pl


def kernel(feat0, feat1, feat2, feat3, bboxes):
    raise NotImplementedError("write your pallas kernel here")


# speedup vs baseline: 1.3421x; 1.3421x over previous
import functools

import jax
import jax.numpy as jnp
from jax import lax
from jax.experimental import pallas as pl
from jax.experimental.pallas import tpu as pltpu
from jax.experimental.pallas import tpu_sc as plsc

_CROP = 7
_OFFSET = 0.5
_LEVEL_SHAPES = ((256, 256), (128, 128), (64, 64), (32, 32))


def _prep(bboxes):
    B, R = bboxes.shape[0], bboxes.shape[1]
    max_h = float(_LEVEL_SHAPES[0][0])
    max_w = float(_LEVEL_SHAPES[0][1])

    bw = bboxes[:, :, 3] - bboxes[:, :, 1]
    bh = bboxes[:, :, 2] - bboxes[:, :, 0]
    areas_sqrt = jnp.sqrt(bh * bw)
    levels = (jnp.floor(jnp.log(areas_sqrt / 224.0) / jnp.log(2.0)) + 4.0).astype(jnp.int32)
    levels = jnp.clip(levels, 1, 4)

    scale = jnp.power(2.0, levels.astype(jnp.float32))
    y0c = bboxes[:, :, 0] / scale
    x0c = bboxes[:, :, 1] / scale
    bh_s = bh / scale
    bw_s = bw / scale

    lvl_idx = levels - 1
    strides = jnp.power(2.0, lvl_idx.astype(jnp.float32))
    bdy = (max_h / strides - 1.0)[..., None]
    bdx = (max_w / strides - 1.0)[..., None]

    grid = (jnp.arange(_CROP, dtype=jnp.float32) + _OFFSET) / float(_CROP)
    gx = x0c[..., None] + grid[None, None, :] * bw_s[..., None]
    gy = y0c[..., None] + grid[None, None, :] * bh_s[..., None]
    x0 = jnp.minimum(jnp.maximum(0.0, jnp.floor(gx)), bdx)
    x1 = jnp.minimum(x0 + 1.0, bdx)
    y0 = jnp.minimum(jnp.maximum(0.0, jnp.floor(gy)), bdy)
    y1 = jnp.minimum(y0 + 1.0, bdy)
    lx = gx - x0
    hx = 1.0 - lx
    ly = gy - y0
    hy = 1.0 - ly

    sizes = [h * w for h, w in _LEVEL_SHAPES]
    offs = [0]
    for s in sizes[:-1]:
        offs.append(offs[-1] + s)
    batch_stride = offs[-1] + sizes[-1]
    widths = jnp.array([w for _, w in _LEVEL_SHAPES], jnp.int32)[lvl_idx]
    lvl_off = jnp.array(offs, jnp.int32)[lvl_idx]
    base = jnp.arange(B, dtype=jnp.int32)[:, None] * batch_stride + lvl_off

    xi0 = x0.astype(jnp.int32)
    xi1 = x1.astype(jnp.int32)
    yi0 = y0.astype(jnp.int32)
    yi1 = y1.astype(jnp.int32)

    def flat(yi, xi):
        return (base[..., None, None]
                + yi[..., :, None] * widths[..., None, None]
                + xi[..., None, :])

    P = _CROP * _CROP
    l0 = jnp.concatenate([flat(yi0, xi0).reshape(B * R, P),
                          flat(yi0, xi1).reshape(B * R, P)], axis=-1)
    l1 = jnp.concatenate([flat(yi1, xi0).reshape(B * R, P),
                          flat(yi1, xi1).reshape(B * R, P)], axis=-1)
    idx = jnp.stack([l0, l1], axis=1)
    idx = jnp.pad(idx, ((0, 0), (0, 0), (0, 104 - 2 * P)))
    wts = jnp.stack([hy, ly, hx, lx], axis=2).reshape(B * R, 4 * _CROP)
    wts = jnp.broadcast_to(wts[:, :, None], (B * R, 4 * _CROP, 16))
    return idx, wts


def _roi_align_sc(table, idx, wts):
    N = idx.shape[0]
    C = table.shape[1]
    nch = C // 16
    P = _CROP * _CROP

    info = plsc.get_sparse_core_info()
    NW = info.num_cores * info.num_subcores
    BPW = N // NW

    mesh = plsc.VectorSubcoreMesh(core_axis_name="c", subcore_axis_name="s")

    @functools.partial(
        pl.kernel,
        mesh=mesh,
        compiler_params=pltpu.CompilerParams(use_tc_tiling_on_sc=False),
        out_type=jax.ShapeDtypeStruct((N, P, C), jnp.float32),
        scratch_types=[
            pltpu.VMEM((BPW, 2, 104), jnp.int32),
            pltpu.VMEM((BPW, 4 * _CROP, 16), jnp.float32),
            pltpu.VMEM((2, 104, C), jnp.float32),
            pltpu.VMEM((P, C), jnp.float32),
            pltpu.SemaphoreType.DMA,
        ],
    )
    def k(table_hbm, idx_hbm, wts_hbm, out_hbm, idx_v, wts_v, rows_v, out_v, gsem):
        wid = lax.axis_index("s") * info.num_cores + lax.axis_index("c")
        first = wid * BPW
        pltpu.sync_copy(idx_hbm.at[pl.ds(first, BPW)], idx_v)
        pltpu.sync_copy(wts_hbm.at[pl.ds(first, BPW)], wts_v)

        def box_body(t, carry):
            cps = [
                pltpu.make_async_copy(
                    table_hbm.at[idx_v.at[t, l]], rows_v.at[l], gsem)
                for l in range(2)
            ]
            for cp in cps:
                cp.start()
            for cp in cps:
                cp.wait()

            def row_body(i, carry2):
                hy = wts_v[t, i, :]
                ly = wts_v[t, _CROP + i, :]

                def col_body(j, carry3):
                    hx = wts_v[t, 2 * _CROP + j, :]
                    lx = wts_v[t, 3 * _CROP + j, :]
                    r = i * _CROP + j
                    for c in range(nch):
                        sl = pl.ds(c * 16, 16)
                        f00 = rows_v[0, r, sl]
                        f01 = rows_v[0, P + r, sl]
                        f10 = rows_v[1, r, sl]
                        f11 = rows_v[1, P + r, sl]
                        out_v[r, sl] = hy * (hx * f00 + lx * f01) + ly * (hx * f10 + lx * f11)
                    return carry3

                return lax.fori_loop(0, _CROP, col_body, carry2)

            lax.fori_loop(0, _CROP, row_body, 0)
            pltpu.sync_copy(out_v, out_hbm.at[first + t])
            return carry

        lax.fori_loop(0, BPW, box_body, 0)

    return k(table, idx, wts)


def kernel(feat0, feat1, feat2, feat3, bboxes):
    B, _, _, C = feat0.shape
    R = bboxes.shape[1]
    table = jnp.concatenate(
        [f.reshape(B, -1, C) for f in (feat0, feat1, feat2, feat3)], axis=1
    ).reshape(-1, C)
    idx, wts = _prep(bboxes)
    out = _roi_align_sc(table, idx, wts)
    return out.reshape(B, R, _CROP, _CROP, C)
